# baseline (speedup 1.0000x reference)
import functools

import jax
import jax.numpy as jnp
from jax import lax
from jax.experimental import pallas as pl
from jax.experimental.pallas import tpu as pltpu
from jax.experimental.pallas import tpu_sc as plsc

VOCAB = 1000000
HIDDEN = 64
BATCH = 16384
SEQ = 50

NC = 2
NS = 16
NW = NC * NS
BPW = BATCH // NW
CHUNK = BPW * SEQ

PR2 = 256
VSEG = VOCAB // 5
VSEGP = VSEG // 2
CORE_P = VSEGP // NC
NPC = 14
HREG = HIDDEN // 16


def _tc_wb(wa_ref, wc_ref, bc_ref, o_ref):
    w2t = lax.dot_general(wc_ref[...], wa_ref[...], (((0,), (1,)), ((), ())),
                          preferred_element_type=jnp.float32)
    w2t = w2t * (1.0 / SEQ)
    b2 = bc_ref[...] * (1.0 / (SEQ * 16))
    brows = jnp.broadcast_to(b2[:, None], (2, HIDDEN))
    o_ref[...] = jnp.concatenate([w2t, brows], axis=0)


def _make_wb(w_a, w_c, b_c):
    return pl.pallas_call(
        _tc_wb,
        out_shape=jax.ShapeDtypeStruct((4, HIDDEN), jnp.float32),
    )(w_a, w_c, b_c)


def _proj_body(voff, table_hbm, wb_hbm, pc0_hbm, pc1_hbm,
               wb_v, ba_v, bb_v, o0_v, o1_v, oka_v, okb_v,
               semi_a, semi_b, semo_a, semo_b):
    c = lax.axis_index("c")
    s = lax.axis_index("s")
    pltpu.sync_copy(wb_hbm, wb_v)

    w0 = [wb_v[0, pl.ds(16 * k, 16)] for k in range(HREG)]
    w1 = [wb_v[1, pl.ds(16 * k, 16)] for k in range(HREG)]
    b0v = wb_v[2, pl.ds(0, 16)]
    b1v = wb_v[3, pl.ds(0, 16)]
    mask15 = lax.iota(jnp.int32, 16) == 15

    ibufs = (ba_v, bb_v)
    isems = (semi_a, semi_b)
    obufs = (oka_v, okb_v)
    osems = (semo_a, semo_b)

    def local_pair(t):
        off = jnp.minimum((s + NS * t) * PR2, CORE_P - PR2)
        return pl.multiple_of(off, 8)

    def src_pair(t):
        return pl.multiple_of(voff // 2 + c * CORE_P + local_pair(t), 8)

    pltpu.async_copy(table_hbm.at[pl.ds(src_pair(0), PR2)], ba_v, semi_a)
    pltpu.async_copy(table_hbm.at[pl.ds(src_pair(1), PR2)], bb_v, semi_b)

    def store_out(buf, t, sem):
        dst = pl.ds(2 * local_pair(t), 2 * PR2)
        @pl.when(c == 0)
        def _():
            pltpu.async_copy(buf, pc0_hbm.at[dst], sem)
        @pl.when(c != 0)
        def _():
            pltpu.async_copy(buf, pc1_hbm.at[dst], sem)

    def wait_out(buf, t, sem):
        dst = pl.ds(2 * local_pair(t), 2 * PR2)
        @pl.when(c == 0)
        def _():
            pltpu.make_async_copy(buf, pc0_hbm.at[dst], sem).wait()
        @pl.when(c != 0)
        def _():
            pltpu.make_async_copy(buf, pc1_hbm.at[dst], sem).wait()

    def project_chunk(buf, opk):
        def group(g, _):
            for rr in range(8):
                r = g * 8 + rr
                ch = [buf[r, pl.ds(16 * k, 16)] for k in range(2 * HREG)]
                p0e = (ch[0] * w0[0] + ch[1] * w0[1]) + \
                      (ch[2] * w0[2] + ch[3] * w0[3]) + b0v
                p1e = (ch[0] * w1[0] + ch[1] * w1[1]) + \
                      (ch[2] * w1[2] + ch[3] * w1[3]) + b1v
                p0o = (ch[4] * w0[0] + ch[5] * w0[1]) + \
                      (ch[6] * w0[2] + ch[7] * w0[3]) + b0v
                p1o = (ch[4] * w1[0] + ch[5] * w1[1]) + \
                      (ch[6] * w1[2] + ch[7] * w1[3]) + b1v
                re = jnp.zeros((16,), jnp.int32) + 2 * r
                ro = re + 1
                plsc.store_scatter(o0_v, [re], plsc.cumsum(p0e), mask=mask15)
                plsc.store_scatter(o1_v, [re], plsc.cumsum(p1e), mask=mask15)
                plsc.store_scatter(o0_v, [ro], plsc.cumsum(p0o), mask=mask15)
                plsc.store_scatter(o1_v, [ro], plsc.cumsum(p1o), mask=mask15)
            return 0
        lax.fori_loop(0, PR2 // 8, group, 0)
        def packg(t, _):
            a = o0_v[pl.ds(16 * t, 16)]
            b = o1_v[pl.ds(16 * t, 16)]
            pk = plsc.bitcast(
                plsc.pack(a, b, format=plsc.PackFormat.INTERLEAVED),
                jnp.int32)
            opk[pl.ds(16 * t, 16)] = pk
            return 0
        lax.fori_loop(0, (2 * PR2) // 16, packg, 0)

    def step(t2, _):
        for b in range(2):
            t = t2 + b
            pltpu.make_async_copy(
                table_hbm.at[pl.ds(src_pair(t), PR2)],
                ibufs[b], isems[b]).wait()
            @pl.when(t >= 2)
            def _():
                wait_out(obufs[b], t - 2, osems[b])
            project_chunk(ibufs[b], obufs[b])
            store_out(obufs[b], t, osems[b])
            @pl.when(t + 2 < NPC)
            def _():
                pltpu.async_copy(
                    table_hbm.at[pl.ds(src_pair(t + 2), PR2)],
                    ibufs[b], isems[b])
        return 0

    lax.fori_loop(0, NPC // 2, lambda i, x: step(i * 2, x), 0)
    for b in range(2):
        wait_out(obufs[b], NPC - 2 + b, osems[b])


def _project_seg(table128, wb, voff):
    mesh = plsc.VectorSubcoreMesh(core_axis_name="c", subcore_axis_name="s")
    fn = pl.kernel(
        functools.partial(_proj_body, voff),
        out_type=[
            pltpu.HBM((2 * CORE_P,), jnp.int32),
            pltpu.HBM((2 * CORE_P,), jnp.int32),
        ],
        mesh=mesh,
        scratch_types=[
            pltpu.VMEM((4, HIDDEN), jnp.float32),
            pltpu.VMEM((PR2, 2 * HIDDEN), jnp.float32),
            pltpu.VMEM((PR2, 2 * HIDDEN), jnp.float32),
            pltpu.VMEM((2 * PR2,), jnp.float32),
            pltpu.VMEM((2 * PR2,), jnp.float32),
            pltpu.VMEM((2 * PR2,), jnp.int32),
            pltpu.VMEM((2 * PR2,), jnp.int32),
            pltpu.SemaphoreType.DMA,
            pltpu.SemaphoreType.DMA,
            pltpu.SemaphoreType.DMA,
            pltpu.SemaphoreType.DMA,
        ],
        compiler_params=pltpu.CompilerParams(needs_layout_passes=False),
    )
    return fn(table128, wb)


def _gather_body(pp_hbm, ids_hbm, out0_hbm, out1_hbm,
                 raw_v, idx_v, g_v, acc0_v, acc1_v, sem):
    c = lax.axis_index("c")
    s = lax.axis_index("s")
    w = c * NS + s
    pltpu.sync_copy(ids_hbm.at[w], raw_v)
    def pbody(j, _):
        jv = lax.iota(jnp.int32, 16) + j * 16
        bb = jnp.bitwise_and(jv, BPW - 1)
        ll = jnp.right_shift(jv, 9)
        src = bb * SEQ + ll
        idx_v[pl.ds(j * 16, 16)] = plsc.load_gather(raw_v, [src])
        return 0
    lax.fori_loop(0, CHUNK // 16, pbody, 0)

    pltpu.async_copy(pp_hbm.at[idx_v], g_v, sem).wait()
    for t in range(BPW // 16):
        def body(l, accs):
            a0, a1 = accs
            pk = g_v[pl.ds(l * BPW + t * 16, 16)]
            b0, b1 = plsc.unpack(plsc.bitcast(pk, jnp.bfloat16),
                                 format=plsc.PackFormat.INTERLEAVED)
            return (a0 + b0.astype(jnp.float32), a1 + b1.astype(jnp.float32))
        init = (jnp.zeros((16,), jnp.float32), jnp.zeros((16,), jnp.float32))
        acc0, acc1 = lax.fori_loop(0, SEQ, body, init)
        acc0_v[pl.ds(t * 16, 16)] = acc0
        acc1_v[pl.ds(t * 16, 16)] = acc1
    pltpu.sync_copy(acc0_v, out0_hbm.at[pl.ds(w * BPW, BPW)])
    pltpu.sync_copy(acc1_v, out1_hbm.at[pl.ds(w * BPW, BPW)])


def _gather_sum(pp, ids_r):
    mesh = plsc.VectorSubcoreMesh(core_axis_name="c", subcore_axis_name="s")
    fn = pl.kernel(
        _gather_body,
        out_type=[
            jax.ShapeDtypeStruct((BATCH,), jnp.float32),
            jax.ShapeDtypeStruct((BATCH,), jnp.float32),
        ],
        mesh=mesh,
        scratch_types=[
            pltpu.VMEM((CHUNK,), jnp.int32),
            pltpu.VMEM((CHUNK,), jnp.int32),
            pltpu.VMEM((CHUNK,), jnp.int32),
            pltpu.VMEM((BPW,), jnp.float32),
            pltpu.VMEM((BPW,), jnp.float32),
            pltpu.SemaphoreType.DMA,
        ],
        compiler_params=pltpu.CompilerParams(needs_layout_passes=False),
    )
    return fn(pp, ids_r)


@jax.jit
def kernel(input_ids, table, W_a, W_c, b_c):
    wb = _make_wb(W_a, W_c, b_c)
    t128 = table.reshape(VOCAB // 2, 2 * HIDDEN)
    parts = []
    for q in range(5):
        pc0, pc1 = _project_seg(t128, wb, q * VSEG)
        parts.extend([pc0, pc1])
        wb = wb + (pc0[0] & 0).astype(jnp.float32)
    pp = jnp.concatenate(parts)
    ids_r = input_ids.astype(jnp.int32).reshape(NW, CHUNK)
    out0, out1 = _gather_sum(pp, ids_r)
    return jnp.stack([out0, out1], axis=1)


# speedup vs baseline: 1.6370x; 1.6370x over previous
import jax
import jax.numpy as jnp
from jax import lax
from jax.experimental import pallas as pl
from jax.experimental.pallas import tpu as pltpu
from jax.experimental.pallas import tpu_sc as plsc

VOCAB = 1000000
HIDDEN = 64
BATCH = 16384
SEQ = 50

NC = 2
NS = 16
NW = NC * NS
BPW = BATCH // NW
CHUNK = BPW * SEQ

TC_BLK = 16384


def _tc_project(w_a_ref, w_c_ref, b_c_ref, t_ref, p0_ref, p1_ref):
    w2 = jnp.dot(w_a_ref[...], w_c_ref[...], preferred_element_type=jnp.float32)
    w2t = w2.T * (1.0 / SEQ)
    t = t_ref[...]
    res = lax.dot_general(w2t, t, (((1,), (1,)), ((), ())),
                          preferred_element_type=jnp.float32)
    b2 = b_c_ref[...] * (1.0 / SEQ)
    p0_ref[...] = res[0:1, :] + b2[0]
    p1_ref[...] = res[1:2, :] + b2[1]


def _project_table(table, w_a, w_c, b_c):
    grid = pl.cdiv(VOCAB, TC_BLK)
    p0, p1 = pl.pallas_call(
        _tc_project,
        grid=(grid,),
        in_specs=[
            pl.BlockSpec((HIDDEN, 3), lambda i: (0, 0)),
            pl.BlockSpec((3, 2), lambda i: (0, 0)),
            pl.BlockSpec((2,), lambda i: (0,)),
            pl.BlockSpec((TC_BLK, HIDDEN), lambda i: (i, 0)),
        ],
        out_specs=[
            pl.BlockSpec((1, TC_BLK), lambda i: (0, i)),
            pl.BlockSpec((1, TC_BLK), lambda i: (0, i)),
        ],
        out_shape=[
            jax.ShapeDtypeStruct((1, VOCAB), jnp.float32),
            jax.ShapeDtypeStruct((1, VOCAB), jnp.float32),
        ],
        compiler_params=pltpu.CompilerParams(
            dimension_semantics=("arbitrary",),
        ),
    )(w_a, w_c, b_c, table)
    return p0.reshape(VOCAB), p1.reshape(VOCAB)


def _accumulate(g_ref, acc_ref):
    for t in range(BPW // 16):
        def body(l, a):
            return a + g_ref[pl.ds(l * BPW + t * 16, 16)]
        acc = lax.fori_loop(0, SEQ, body, jnp.zeros((16,), jnp.float32))
        acc_ref[pl.ds(t * 16, 16)] = acc


def _sc_body(p0_hbm, p1_hbm, ids_hbm, out0_hbm, out1_hbm,
             idx_v, g0_v, g1_v, acc_v, sem0, sem1):
    c = lax.axis_index("c")
    s = lax.axis_index("s")
    w = c * NS + s
    pltpu.sync_copy(ids_hbm.at[w], idx_v)
    cp0 = pltpu.async_copy(p0_hbm.at[idx_v], g0_v, sem0)
    cp1 = pltpu.async_copy(p1_hbm.at[idx_v], g1_v, sem1)
    cp0.wait()
    _accumulate(g0_v, acc_v)
    pltpu.sync_copy(acc_v, out0_hbm.at[pl.ds(w * BPW, BPW)])
    cp1.wait()
    _accumulate(g1_v, acc_v)
    pltpu.sync_copy(acc_v, out1_hbm.at[pl.ds(w * BPW, BPW)])


def _gather_sum(p0, p1, ids_r):
    mesh = plsc.VectorSubcoreMesh(core_axis_name="c", subcore_axis_name="s")
    fn = pl.kernel(
        _sc_body,
        out_type=[
            jax.ShapeDtypeStruct((BATCH,), jnp.float32),
            jax.ShapeDtypeStruct((BATCH,), jnp.float32),
        ],
        mesh=mesh,
        scratch_types=[
            pltpu.VMEM((CHUNK,), jnp.int32),
            pltpu.VMEM((CHUNK,), jnp.float32),
            pltpu.VMEM((CHUNK,), jnp.float32),
            pltpu.VMEM((BPW,), jnp.float32),
            pltpu.SemaphoreType.DMA,
            pltpu.SemaphoreType.DMA,
        ],
    )
    return fn(p0, p1, ids_r)


@jax.jit
def kernel(input_ids, table, W_a, W_c, b_c):
    p0, p1 = _project_table(table, W_a, W_c, b_c)
    ids_r = (input_ids.astype(jnp.int32)
             .reshape(NW, BPW, SEQ)
             .transpose(0, 2, 1)
             .reshape(NW, CHUNK))
    out0, out1 = _gather_sum(p0, p1, ids_r)
    return jnp.stack([out0, out1], axis=1)
